# Initial kernel scaffold; baseline (speedup 1.0000x reference)
import jax
import jax.numpy as jnp
from jax.experimental import pallas as pl


def kernel(batch_token, table):
    raise NotImplementedError("write your pallas kernel here")


# speedup vs baseline: 5.1091x; 5.1091x over previous
import functools
import math

import jax
import jax.numpy as jnp
import numpy as np
from jax.experimental import pallas as pl
from jax.experimental.pallas import tpu as pltpu

_PAD_IDX = 0
_BB = 8


@functools.lru_cache(maxsize=None)
def _make_pe(seq: int, d_model: int):
    position = np.arange(seq, dtype=np.float64)[:, None]
    div_term = np.exp(
        np.arange(0, d_model, 2, dtype=np.float64) * -(math.log(10000.0) / d_model)
    )
    pe = np.zeros((seq, d_model), dtype=np.float64)
    pe[:, 0::2] = np.sin(position * div_term)
    pe[:, 1::2] = np.cos(position * div_term)
    return jnp.asarray(pe, dtype=jnp.float32)


def _embed_kernel(tok_ref, table_ref, pe_ref, x_ref, mask_ref):
    _, s, bb = tok_ref.shape
    vocab, _ = table_ref.shape
    table = table_ref[...]
    pe = pe_ref[...]
    tok_t = tok_ref[0]
    iota_v = jax.lax.broadcasted_iota(jnp.int32, (1, vocab), 1)
    for r in range(bb):
        col = tok_t[:, r : r + 1]
        onehot = (col == iota_v).astype(jnp.float32)
        x_ref[r] = jnp.dot(onehot, table, preferred_element_type=jnp.float32) + pe
        m = (col != _PAD_IDX).astype(jnp.float32)
        mask_ref[r] = jnp.broadcast_to(m, (s, s))


def kernel(batch_token, table):
    b, s = batch_token.shape
    vocab, d = table.shape
    pe = _make_pe(s, d)
    nb = b // _BB
    tok3 = batch_token.reshape(nb, _BB, s).transpose(0, 2, 1)
    x, mask = pl.pallas_call(
        _embed_kernel,
        grid=(nb,),
        in_specs=[
            pl.BlockSpec((1, s, _BB), lambda i: (i, 0, 0)),
            pl.BlockSpec((vocab, d), lambda i: (0, 0)),
            pl.BlockSpec((s, d), lambda i: (0, 0)),
        ],
        out_specs=[
            pl.BlockSpec((_BB, s, d), lambda i: (i, 0, 0)),
            pl.BlockSpec((_BB, s, s), lambda i: (i, 0, 0)),
        ],
        out_shape=[
            jax.ShapeDtypeStruct((b, s, d), jnp.float32),
            jax.ShapeDtypeStruct((b, s, s), jnp.float32),
        ],
    )(tok3, table, pe)
    return (x, mask)


# speedup vs baseline: 5.1518x; 1.0084x over previous
import functools
import math

import jax
import jax.numpy as jnp
import numpy as np
from jax.experimental import pallas as pl
from jax.experimental.pallas import tpu as pltpu

_PAD_IDX = 0
_BB = 16


@functools.lru_cache(maxsize=None)
def _make_pe(seq: int, d_model: int):
    position = np.arange(seq, dtype=np.float64)[:, None]
    div_term = np.exp(
        np.arange(0, d_model, 2, dtype=np.float64) * -(math.log(10000.0) / d_model)
    )
    pe = np.zeros((seq, d_model), dtype=np.float64)
    pe[:, 0::2] = np.sin(position * div_term)
    pe[:, 1::2] = np.cos(position * div_term)
    return jnp.asarray(pe, dtype=jnp.float32)


def _embed_kernel(tok_ref, table_ref, pe_ref, x_ref, mask_ref):
    _, s, bb = tok_ref.shape
    vocab, _ = table_ref.shape
    table = table_ref[...]
    pe = pe_ref[...]
    tok_t = tok_ref[0]
    iota_v = jax.lax.broadcasted_iota(jnp.int32, (1, vocab), 1)
    for r in range(bb):
        col = tok_t[:, r : r + 1]
        onehot = (col == iota_v).astype(jnp.float32)
        x_ref[r] = jnp.dot(onehot, table, preferred_element_type=jnp.float32) + pe
        m = (col != _PAD_IDX).astype(jnp.float32)
        mask_ref[r] = jnp.broadcast_to(m, (s, s))


def kernel(batch_token, table):
    b, s = batch_token.shape
    vocab, d = table.shape
    pe = _make_pe(s, d)
    nb = b // _BB
    tok3 = batch_token.reshape(nb, _BB, s).transpose(0, 2, 1)
    x, mask = pl.pallas_call(
        _embed_kernel,
        grid=(nb,),
        in_specs=[
            pl.BlockSpec((1, s, _BB), lambda i: (i, 0, 0)),
            pl.BlockSpec((vocab, d), lambda i: (0, 0)),
            pl.BlockSpec((s, d), lambda i: (0, 0)),
        ],
        out_specs=[
            pl.BlockSpec((_BB, s, d), lambda i: (i, 0, 0)),
            pl.BlockSpec((_BB, s, s), lambda i: (i, 0, 0)),
        ],
        out_shape=[
            jax.ShapeDtypeStruct((b, s, d), jnp.float32),
            jax.ShapeDtypeStruct((b, s, s), jnp.float32),
        ],
    )(tok3, table, pe)
    return (x, mask)
